# baseline (speedup 1.0000x reference)
import functools

import jax
import jax.numpy as jnp
from jax import lax
from jax.experimental import pallas as pl
from jax.experimental.pallas import tpu as pltpu
from jax.experimental.pallas import tpu_sc as plsc

BATCH = 16384
EMBED = 32
NC = 2
NS = 16
L = 16
NW = NC * NS
BPW = BATCH // NW
CHUNK = 128
NCHUNK = BPW // CHUNK
GROUPS = BPW // L


def _rsqrt16(x):
    i = plsc.bitcast(x, jnp.int32)
    i = jnp.int32(0x5F3759DF) - lax.shift_right_logical(i, 1)
    y = plsc.bitcast(i, jnp.float32)
    xh = x * jnp.float32(0.5)
    for _ in range(3):
        y = y * (jnp.float32(1.5) - xh * y * y)
    return y


def _make_kernel():
    mesh = plsc.VectorSubcoreMesh(
        core_axis_name="c", subcore_axis_name="s",
        num_cores=NC, num_subcores=NS)

    @functools.partial(
        pl.kernel,
        out_type=jax.ShapeDtypeStruct((BATCH, EMBED), jnp.float32),
        mesh=mesh,
        scratch_types=[
            pltpu.VMEM((NCHUNK, CHUNK), jnp.int32),
            pltpu.VMEM((BPW, EMBED), jnp.float32),
            pltpu.SemaphoreType.DMA,
        ],
        compiler_params=pltpu.CompilerParams(
            needs_layout_passes=False, use_tc_tiling_on_sc=False),
    )
    def sc_embed_norm(idx_hbm, table_hbm, out_hbm, idx_v, rows_v, sem):
        wid = lax.axis_index("s") * NC + lax.axis_index("c")
        base = wid * BPW

        pltpu.sync_copy(idx_hbm.at[pl.ds(wid * NCHUNK, NCHUNK)], idx_v)
        for j in range(NCHUNK):
            for k in range(CHUNK // L):
                sl = (j, pl.ds(k * L, L))
                idx_v[sl] = idx_v[sl] + 1

        copies = [
            pltpu.async_copy(
                table_hbm.at[idx_v.at[j]],
                rows_v.at[pl.ds(j * CHUNK, CHUNK)],
                sem)
            for j in range(NCHUNK)
        ]
        for c in copies:
            c.wait()

        @plsc.parallel_loop(0, BPW, unroll=4)
        def norm_row(i):
            a = rows_v[i, pl.ds(0, L)]
            b = rows_v[i, pl.ds(L, L)]
            s2 = jnp.sum(a * a + b * b)
            s2v = jnp.full((L,), s2, jnp.float32)
            r = _rsqrt16(jnp.maximum(s2v, jnp.float32(1e-12)))
            rows_v[i, pl.ds(0, L)] = a * r
            rows_v[i, pl.ds(L, L)] = b * r

        pltpu.sync_copy(rows_v, out_hbm.at[pl.ds(base, BPW)])

    return sc_embed_norm


_KERNEL = _make_kernel()


def kernel(customer_id, embedding_table):
    idx = customer_id.reshape(NW * NCHUNK, CHUNK)
    return _KERNEL(idx, embedding_table)
